# Initial kernel scaffold; baseline (speedup 1.0000x reference)
import jax
import jax.numpy as jnp
from jax.experimental import pallas as pl


def kernel(x, Wg, bg, W1, b1, W2, b2):
    raise NotImplementedError("write your pallas kernel here")


# speedup vs baseline: 2.2299x; 2.2299x over previous
import functools

import jax
import jax.numpy as jnp
from jax.experimental import pallas as pl
from jax.experimental.pallas import tpu as pltpu

S, D = 2048, 768
E, K, H = 8, 2, 1536
SB = 256
HC = 512


def _moe_dense_kernel(x_ref, wg_ref, bg_ref, w1_ref, b1_ref, w2_ref, b2_ref,
                      out_ref, gate_ref, c_ref):
    j = pl.program_id(0)

    @pl.when(j == 0)
    def _router():
        xv = x_ref[...]
        logits = jnp.dot(xv, wg_ref[...], preferred_element_type=jnp.float32)
        logits = logits + bg_ref[...]
        m = jnp.max(logits, axis=1, keepdims=True)
        ex = jnp.exp(logits - m)
        probs = ex / jnp.sum(ex, axis=1, keepdims=True)
        gate_ref[...] = probs
        idx = jax.lax.broadcasted_iota(jnp.int32, (S, E), 1)
        p1 = jnp.max(probs, axis=1, keepdims=True)
        i1 = jnp.min(jnp.where(probs == p1, idx, E), axis=1, keepdims=True)
        pm = jnp.where(idx == i1, -jnp.inf, probs)
        p2 = jnp.max(pm, axis=1, keepdims=True)
        i2 = jnp.min(jnp.where(pm == p2, idx, E), axis=1, keepdims=True)
        t = jnp.exp(p2 - p1)
        q1 = 1.0 / (1.0 + t)
        q2 = t / (1.0 + t)
        c_ref[...] = jnp.where(idx == i1, q1, 0.0) + jnp.where(idx == i2, q2, 0.0)
        out_ref[...] = jnp.zeros((S, D), jnp.float32)

    def body(i, _):
        sl = pl.ds(i * SB, SB)
        xv = x_ref[sl, :]
        acc = jnp.broadcast_to(b2_ref[0], (SB, D))
        for hc in range(H // HC):
            w1c = w1_ref[0, :, hc * HC:(hc + 1) * HC]
            h = jnp.maximum(
                jnp.dot(xv, w1c, preferred_element_type=jnp.float32)
                + b1_ref[0, :, hc * HC:(hc + 1) * HC], 0.0)
            acc = acc + jnp.dot(h, w2_ref[0, hc * HC:(hc + 1) * HC, :],
                                preferred_element_type=jnp.float32)
        cb = c_ref[sl, :]
        lane = jax.lax.broadcasted_iota(jnp.int32, (SB, E), 1)
        cj = jnp.sum(jnp.where(lane == j, cb, 0.0), axis=1, keepdims=True)
        out_ref[sl, :] += cj * acc
        return 0

    jax.lax.fori_loop(0, S // SB, body, 0)


@functools.partial(jax.jit, static_argnames=())
def kernel(x, Wg, bg, W1, b1, W2, b2):
    B = x.shape[0]
    xs = x.reshape(S, D)
    out, gate = pl.pallas_call(
        _moe_dense_kernel,
        grid=(E,),
        in_specs=[
            pl.BlockSpec((S, D), lambda j: (0, 0)),
            pl.BlockSpec((D, E), lambda j: (0, 0)),
            pl.BlockSpec((1, E), lambda j: (0, 0)),
            pl.BlockSpec((1, D, H), lambda j: (j, 0, 0)),
            pl.BlockSpec((1, 1, H), lambda j: (j, 0, 0)),
            pl.BlockSpec((1, H, D), lambda j: (j, 0, 0)),
            pl.BlockSpec((1, 1, D), lambda j: (j, 0, 0)),
        ],
        out_specs=[
            pl.BlockSpec((S, D), lambda j: (0, 0)),
            pl.BlockSpec((S, E), lambda j: (0, 0)),
        ],
        out_shape=[
            jax.ShapeDtypeStruct((S, D), jnp.float32),
            jax.ShapeDtypeStruct((S, E), jnp.float32),
        ],
        scratch_shapes=[pltpu.VMEM((S, E), jnp.float32)],
    )(xs, Wg, bg.reshape(1, E), W1, b1.reshape(E, 1, H), W2, b2.reshape(E, 1, D))
    return out.reshape(B, S, D), gate.reshape(B, S, E)
